# Initial kernel scaffold; baseline (speedup 1.0000x reference)
import functools

import jax
import jax.numpy as jnp
from jax import lax
from jax.experimental import pallas as pl
from jax.experimental.pallas import tpu as pltpu
from jax.experimental.pallas import tpu_sc as plsc

N = 10000
E = 320000
DIN = 128
DH = 64
DOUT = 2

RN = 10240
JUNK = N
DAUG = 80
D2 = 16
NSC = 2
NTILE = 16
NW = NSC * NTILE
CHUNK = 128
CPT = 79
NCH = NW * CPT
EPAD = NCH * CHUNK

BR = 512
GRID = RN // BR


def _tc1_body(x_ref, wl_ref, wr_ref, bl_ref, br_ref, aug_ref, xr_ref):
    xv = x_ref[...]
    aug_ref[...] = jnp.dot(xv, wl_ref[...],
                           preferred_element_type=jnp.float32) + bl_ref[...]
    xr_ref[...] = jnp.dot(xv, wr_ref[...],
                          preferred_element_type=jnp.float32) + br_ref[...]


def _tc1(xp, wl_aug, w1r, bl, br):
    return pl.pallas_call(
        _tc1_body,
        grid=(GRID,),
        in_specs=[
            pl.BlockSpec((BR, DIN), lambda i: (i, 0)),
            pl.BlockSpec((DIN, DAUG), lambda i: (0, 0)),
            pl.BlockSpec((DIN, DH), lambda i: (0, 0)),
            pl.BlockSpec((1, DAUG), lambda i: (0, 0)),
            pl.BlockSpec((1, DH), lambda i: (0, 0)),
        ],
        out_specs=[
            pl.BlockSpec((BR, DAUG), lambda i: (i, 0)),
            pl.BlockSpec((BR, DH), lambda i: (i, 0)),
        ],
        out_shape=[
            jax.ShapeDtypeStruct((RN, DAUG), jnp.float32),
            jax.ShapeDtypeStruct((RN, DH), jnp.float32),
        ],
    )(xp, wl_aug, w1r, bl, br)


def _tc2_body(p_ref, xr_ref, wl_ref, wr_ref, b2_ref, hl_ref, hr_ref, deg_ref):
    pv = p_ref[...]
    ssum = pv[0] + pv[1]
    deg = jnp.maximum(ssum[:, DH:DH + 1], 1.0)
    h = jnp.maximum(ssum[:, :DH] / deg + xr_ref[...], 0.0)
    hl_ref[...] = jnp.dot(h, wl_ref[...], preferred_element_type=jnp.float32)
    hr_ref[...] = jnp.dot(h, wr_ref[...],
                          preferred_element_type=jnp.float32) + b2_ref[...]
    deg_ref[...] = jnp.broadcast_to(deg, (BR, D2))


def _tc2(p1, xr, w2l, w2r, b2p):
    return pl.pallas_call(
        _tc2_body,
        grid=(GRID,),
        in_specs=[
            pl.BlockSpec((2, BR, DAUG), lambda i: (0, i, 0)),
            pl.BlockSpec((BR, DH), lambda i: (i, 0)),
            pl.BlockSpec((DH, D2), lambda i: (0, 0)),
            pl.BlockSpec((DH, D2), lambda i: (0, 0)),
            pl.BlockSpec((1, D2), lambda i: (0, 0)),
        ],
        out_specs=[
            pl.BlockSpec((BR, D2), lambda i: (i, 0)),
            pl.BlockSpec((BR, D2), lambda i: (i, 0)),
            pl.BlockSpec((BR, D2), lambda i: (i, 0)),
        ],
        out_shape=[
            jax.ShapeDtypeStruct((RN, D2), jnp.float32),
            jax.ShapeDtypeStruct((RN, D2), jnp.float32),
            jax.ShapeDtypeStruct((RN, D2), jnp.float32),
        ],
    )(p1, xr, w2l, w2r, b2p)


def _tc3_body(p_ref, deg_ref, hr_ref, out_ref):
    pv = p_ref[...]
    out_ref[...] = (pv[0] + pv[1]) / deg_ref[...] + hr_ref[...]


def _tc3(p2, deg, hr):
    return pl.pallas_call(
        _tc3_body,
        grid=(GRID,),
        in_specs=[
            pl.BlockSpec((2, BR, D2), lambda i: (0, i, 0)),
            pl.BlockSpec((BR, D2), lambda i: (i, 0)),
            pl.BlockSpec((BR, D2), lambda i: (i, 0)),
        ],
        out_specs=pl.BlockSpec((BR, D2), lambda i: (i, 0)),
        out_shape=jax.ShapeDtypeStruct((RN, D2), jnp.float32),
    )(p2, deg, hr)


def _make_sc_agg(ncols):
    mesh = plsc.VectorSubcoreMesh(core_axis_name="c", subcore_axis_name="s")
    rpt = RN // NTILE

    @functools.partial(
        pl.kernel,
        mesh=mesh,
        out_type=jax.ShapeDtypeStruct((NSC, RN, ncols), jnp.float32),
        scratch_types=[
            pltpu.VMEM((CPT, CHUNK), jnp.int32),
            pltpu.VMEM((CPT, CHUNK), jnp.int32),
            pltpu.VMEM((CHUNK, ncols), jnp.float32),
            pltpu.VMEM_SHARED((RN, ncols), jnp.float32),
            pltpu.SemaphoreType.DMA,
        ],
    )
    def k(feat, srcs, dsts, zeros, out, srcb, dstb, rows, acc, sem):
        c = lax.axis_index("c")
        s = lax.axis_index("s")
        wid = s * NSC + c
        pltpu.sync_copy(zeros.at[pl.ds(s * rpt, rpt)],
                        acc.at[pl.ds(s * rpt, rpt)])
        pltpu.sync_copy(srcs.at[pl.ds(wid * CPT, CPT)], srcb)
        pltpu.sync_copy(dsts.at[pl.ds(wid * CPT, CPT)], dstb)
        plsc.subcore_barrier()

        def body(i, carry):
            pltpu.async_copy(feat.at[srcb.at[i]], rows, sem).wait()
            pltpu.sync_copy(rows, acc.at[dstb.at[i]], add=True)
            return carry

        lax.fori_loop(0, CPT, body, 0)
        plsc.subcore_barrier()
        pltpu.sync_copy(acc.at[pl.ds(s * rpt, rpt)],
                        out.at[c, pl.ds(s * rpt, rpt)])

    return k


_sc_agg80 = _make_sc_agg(DAUG)
_sc_agg16 = _make_sc_agg(D2)


def kernel(x, edge_index, W1_l, W1_r, b1, W2_l, W2_r, b2):
    src = edge_index[0].astype(jnp.int32)
    dst = edge_index[1].astype(jnp.int32)
    srcp = jnp.zeros((EPAD,), jnp.int32).at[:E].set(src).reshape(NCH, CHUNK)
    dstp = jnp.full((EPAD,), JUNK, jnp.int32).at[:E].set(dst).reshape(NCH, CHUNK)
    xp = jnp.zeros((RN, DIN), jnp.float32).at[:N].set(x)

    wl_aug = jnp.zeros((DIN, DAUG), jnp.float32).at[:, :DH].set(W1_l)
    bl = jnp.zeros((1, DAUG), jnp.float32).at[0, DH].set(1.0)
    brow = b1[None, :]
    aug, xr = _tc1(xp, wl_aug, W1_r, bl, brow)

    zeros1 = jnp.zeros((RN, DAUG), jnp.float32)
    p1 = _sc_agg80(aug, srcp, dstp, zeros1)

    w2l = jnp.zeros((DH, D2), jnp.float32).at[:, :DOUT].set(W2_l)
    w2r = jnp.zeros((DH, D2), jnp.float32).at[:, :DOUT].set(W2_r)
    b2p = jnp.zeros((1, D2), jnp.float32).at[0, :DOUT].set(b2)
    hl, hr, deg = _tc2(p1, xr, w2l, w2r, b2p)

    zeros2 = jnp.zeros((RN, D2), jnp.float32)
    p2 = _sc_agg16(hl, srcp, dstp, zeros2)

    outp = _tc3(p2, deg, hr)
    return outp[:N, :DOUT]


# speedup vs baseline: 6.2753x; 6.2753x over previous
import functools

import jax
import jax.numpy as jnp
from jax import lax
from jax.experimental import pallas as pl
from jax.experimental.pallas import tpu as pltpu
from jax.experimental.pallas import tpu_sc as plsc

N = 10000
E = 320000
DIN = 128
DH = 64
DOUT = 2

RN = 10240
JUNK = N
D2 = 16
NSC = 2
NTILE = 16
NW = NSC * NTILE
CHUNK = 128
CPT = 80
NCH = NW * CPT
EPAD = NCH * CHUNK

BR = 512
GRID = RN // BR


def _tc1_body(x_ref, wl_ref, wr_ref, br_ref, xl_ref, xr_ref):
    xv = x_ref[...]
    xl_ref[...] = jnp.dot(xv, wl_ref[...], preferred_element_type=jnp.float32)
    xr_ref[...] = jnp.dot(xv, wr_ref[...],
                          preferred_element_type=jnp.float32) + br_ref[...]


def _tc1(xp, w1l, w1r, brow):
    return pl.pallas_call(
        _tc1_body,
        grid=(GRID,),
        in_specs=[
            pl.BlockSpec((BR, DIN), lambda i: (i, 0)),
            pl.BlockSpec((DIN, DH), lambda i: (0, 0)),
            pl.BlockSpec((DIN, DH), lambda i: (0, 0)),
            pl.BlockSpec((1, DH), lambda i: (0, 0)),
        ],
        out_specs=[
            pl.BlockSpec((BR, DH), lambda i: (i, 0)),
            pl.BlockSpec((BR, DH), lambda i: (i, 0)),
        ],
        out_shape=[
            jax.ShapeDtypeStruct((RN, DH), jnp.float32),
            jax.ShapeDtypeStruct((RN, DH), jnp.float32),
        ],
    )(xp, w1l, w1r, brow)


def _tc2_body(p_ref, pd_ref, xr_ref, wl_ref, wr_ref, b2_ref,
              hl_ref, hr_ref, deg_ref):
    pv = p_ref[...]
    pd = pd_ref[...]
    ssum = pv[0] + pv[1]
    deg = jnp.maximum(pd[0, :, 0:1] + pd[1, :, 0:1], 1.0)
    h = jnp.maximum(ssum / deg + xr_ref[...], 0.0)
    hl_ref[...] = jnp.dot(h, wl_ref[...], preferred_element_type=jnp.float32)
    hr_ref[...] = jnp.dot(h, wr_ref[...],
                          preferred_element_type=jnp.float32) + b2_ref[...]
    deg_ref[...] = jnp.broadcast_to(deg, (BR, D2))


def _tc2(p1, pdeg, xr, w2l, w2r, b2p):
    return pl.pallas_call(
        _tc2_body,
        grid=(GRID,),
        in_specs=[
            pl.BlockSpec((2, BR, DH), lambda i: (0, i, 0)),
            pl.BlockSpec((2, BR, D2), lambda i: (0, i, 0)),
            pl.BlockSpec((BR, DH), lambda i: (i, 0)),
            pl.BlockSpec((DH, D2), lambda i: (0, 0)),
            pl.BlockSpec((DH, D2), lambda i: (0, 0)),
            pl.BlockSpec((1, D2), lambda i: (0, 0)),
        ],
        out_specs=[
            pl.BlockSpec((BR, D2), lambda i: (i, 0)),
            pl.BlockSpec((BR, D2), lambda i: (i, 0)),
            pl.BlockSpec((BR, D2), lambda i: (i, 0)),
        ],
        out_shape=[
            jax.ShapeDtypeStruct((RN, D2), jnp.float32),
            jax.ShapeDtypeStruct((RN, D2), jnp.float32),
            jax.ShapeDtypeStruct((RN, D2), jnp.float32),
        ],
    )(p1, pdeg, xr, w2l, w2r, b2p)


def _tc3_body(p_ref, deg_ref, hr_ref, out_ref):
    pv = p_ref[...]
    out_ref[...] = (pv[0] + pv[1]) / deg_ref[...] + hr_ref[...]


def _tc3(p2, deg, hr):
    return pl.pallas_call(
        _tc3_body,
        grid=(GRID,),
        in_specs=[
            pl.BlockSpec((2, BR, D2), lambda i: (0, i, 0)),
            pl.BlockSpec((BR, D2), lambda i: (i, 0)),
            pl.BlockSpec((BR, D2), lambda i: (i, 0)),
        ],
        out_specs=pl.BlockSpec((BR, D2), lambda i: (i, 0)),
        out_shape=jax.ShapeDtypeStruct((RN, D2), jnp.float32),
    )(p2, deg, hr)


_MESH = plsc.VectorSubcoreMesh(core_axis_name="c", subcore_axis_name="s")
_RPT = RN // NTILE


@functools.partial(
    pl.kernel,
    mesh=_MESH,
    compiler_params=pltpu.CompilerParams(use_tc_tiling_on_sc=False),
    out_type=[
        jax.ShapeDtypeStruct((NSC, RN, DH), jnp.float32),
        jax.ShapeDtypeStruct((NSC, RN, D2), jnp.float32),
    ],
    scratch_types=[
        pltpu.VMEM((CPT, CHUNK), jnp.int32),
        pltpu.VMEM((CPT, CHUNK), jnp.int32),
        pltpu.VMEM((CHUNK, DH), jnp.float32),
        pltpu.VMEM((CHUNK, D2), jnp.float32),
        pltpu.VMEM_SHARED((RN, DH), jnp.float32),
        pltpu.VMEM_SHARED((RN, D2), jnp.float32),
        pltpu.SemaphoreType.DMA,
    ],
)
def _sc_agg1(feat, srcs, dsts, zeros64, zeros16, ones16,
             out, outdeg, srcb, dstb, rows, onesb, acc, accd, sem):
    c = lax.axis_index("c")
    s = lax.axis_index("s")
    wid = s * NSC + c
    pltpu.sync_copy(zeros64.at[pl.ds(s * _RPT, _RPT)],
                    acc.at[pl.ds(s * _RPT, _RPT)])
    pltpu.sync_copy(zeros16.at[pl.ds(s * _RPT, _RPT)],
                    accd.at[pl.ds(s * _RPT, _RPT)])
    pltpu.sync_copy(ones16, onesb)
    pltpu.sync_copy(srcs.at[pl.ds(wid * CPT, CPT)], srcb)
    pltpu.sync_copy(dsts.at[pl.ds(wid * CPT, CPT)], dstb)
    plsc.subcore_barrier()

    def body(i, carry):
        pltpu.async_copy(feat.at[srcb.at[i]], rows, sem).wait()
        pltpu.sync_copy(rows, acc.at[dstb.at[i]], add=True)
        pltpu.sync_copy(onesb, accd.at[dstb.at[i]], add=True)
        return carry

    lax.fori_loop(0, CPT, body, 0)
    plsc.subcore_barrier()
    pltpu.sync_copy(acc.at[pl.ds(s * _RPT, _RPT)],
                    out.at[c, pl.ds(s * _RPT, _RPT)])
    pltpu.sync_copy(accd.at[pl.ds(s * _RPT, _RPT)],
                    outdeg.at[c, pl.ds(s * _RPT, _RPT)])


@functools.partial(
    pl.kernel,
    mesh=_MESH,
    compiler_params=pltpu.CompilerParams(use_tc_tiling_on_sc=False),
    out_type=jax.ShapeDtypeStruct((NSC, RN, D2), jnp.float32),
    scratch_types=[
        pltpu.VMEM((CPT, CHUNK), jnp.int32),
        pltpu.VMEM((CPT, CHUNK), jnp.int32),
        pltpu.VMEM((CHUNK, D2), jnp.float32),
        pltpu.VMEM_SHARED((RN, D2), jnp.float32),
        pltpu.SemaphoreType.DMA,
    ],
)
def _sc_agg2(feat, srcs, dsts, zeros16, out, srcb, dstb, rows, acc, sem):
    c = lax.axis_index("c")
    s = lax.axis_index("s")
    wid = s * NSC + c
    pltpu.sync_copy(zeros16.at[pl.ds(s * _RPT, _RPT)],
                    acc.at[pl.ds(s * _RPT, _RPT)])
    pltpu.sync_copy(srcs.at[pl.ds(wid * CPT, CPT)], srcb)
    pltpu.sync_copy(dsts.at[pl.ds(wid * CPT, CPT)], dstb)
    plsc.subcore_barrier()

    def body(i, carry):
        pltpu.async_copy(feat.at[srcb.at[i]], rows, sem).wait()
        pltpu.sync_copy(rows, acc.at[dstb.at[i]], add=True)
        return carry

    lax.fori_loop(0, CPT, body, 0)
    plsc.subcore_barrier()
    pltpu.sync_copy(acc.at[pl.ds(s * _RPT, _RPT)],
                    out.at[c, pl.ds(s * _RPT, _RPT)])


def kernel(x, edge_index, W1_l, W1_r, b1, W2_l, W2_r, b2):
    src = edge_index[0].astype(jnp.int32)
    dst = edge_index[1].astype(jnp.int32)
    srcp = jnp.zeros((EPAD,), jnp.int32).at[:E].set(src).reshape(NCH, CHUNK)
    dstp = jnp.full((EPAD,), JUNK, jnp.int32).at[:E].set(dst).reshape(NCH, CHUNK)
    xp = jnp.zeros((RN, DIN), jnp.float32).at[:N].set(x)

    xl, xr = _tc1(xp, W1_l, W1_r, b1[None, :])

    zeros64 = jnp.zeros((RN, DH), jnp.float32)
    zeros16 = jnp.zeros((RN, D2), jnp.float32)
    ones16 = jnp.ones((CHUNK, D2), jnp.float32)
    p1, pdeg = _sc_agg1(xl, srcp, dstp, zeros64, zeros16, ones16)

    w2l = jnp.zeros((DH, D2), jnp.float32).at[:, :DOUT].set(W2_l)
    w2r = jnp.zeros((DH, D2), jnp.float32).at[:, :DOUT].set(W2_r)
    b2p = jnp.zeros((1, D2), jnp.float32).at[0, :DOUT].set(b2)
    hl, hr, deg = _tc2(p1, pdeg, xr, w2l, w2r, b2p)

    p2 = _sc_agg2(hl, srcp, dstp, zeros16)

    outp = _tc3(p2, deg, hr)
    return outp[:N, :DOUT]


# speedup vs baseline: 8.0132x; 1.2770x over previous
import functools

import jax
import jax.numpy as jnp
from jax import lax
from jax.experimental import pallas as pl
from jax.experimental.pallas import tpu as pltpu
from jax.experimental.pallas import tpu_sc as plsc

N = 10000
E = 320000
DIN = 128
DH = 64
DOUT = 2

RN = 10240
JUNK = N
D2 = 16
NSC = 2
NTILE = 16
NW = NSC * NTILE
CHUNK = 128
CPT = 80
NCH = NW * CPT
EPAD = NCH * CHUNK

BR = 512
GRID = RN // BR


def _tc1_body(x_ref, wl_ref, wr_ref, br_ref, xl_ref, xr_ref):
    xv = x_ref[...]
    xl_ref[...] = jnp.dot(xv, wl_ref[...], preferred_element_type=jnp.float32)
    xr_ref[...] = jnp.dot(xv, wr_ref[...],
                          preferred_element_type=jnp.float32) + br_ref[...]


def _tc1(xp, w1l, w1r, brow):
    return pl.pallas_call(
        _tc1_body,
        grid=(GRID,),
        in_specs=[
            pl.BlockSpec((BR, DIN), lambda i: (i, 0)),
            pl.BlockSpec((DIN, DH), lambda i: (0, 0)),
            pl.BlockSpec((DIN, DH), lambda i: (0, 0)),
            pl.BlockSpec((1, DH), lambda i: (0, 0)),
        ],
        out_specs=[
            pl.BlockSpec((BR, DH), lambda i: (i, 0)),
            pl.BlockSpec((BR, DH), lambda i: (i, 0)),
        ],
        out_shape=[
            jax.ShapeDtypeStruct((RN, DH), jnp.float32),
            jax.ShapeDtypeStruct((RN, DH), jnp.float32),
        ],
    )(xp, w1l, w1r, brow)


def _tc2_body(p_ref, pd_ref, xr_ref, wl_ref, wr_ref, b2_ref,
              hl_ref, hr_ref, deg_ref):
    pv = p_ref[...]
    pd = pd_ref[...]
    ssum = pv[0] + pv[1]
    deg = jnp.maximum(pd[0, :, 0:1] + pd[1, :, 0:1], 1.0)
    h = jnp.maximum(ssum / deg + xr_ref[...], 0.0)
    hl_ref[...] = jnp.dot(h, wl_ref[...], preferred_element_type=jnp.float32)
    hr_ref[...] = jnp.dot(h, wr_ref[...],
                          preferred_element_type=jnp.float32) + b2_ref[...]
    deg_ref[...] = jnp.broadcast_to(deg, (BR, D2))


def _tc2(p1, pdeg, xr, w2l, w2r, b2p):
    return pl.pallas_call(
        _tc2_body,
        grid=(GRID,),
        in_specs=[
            pl.BlockSpec((2, BR, DH), lambda i: (0, i, 0)),
            pl.BlockSpec((2, BR, D2), lambda i: (0, i, 0)),
            pl.BlockSpec((BR, DH), lambda i: (i, 0)),
            pl.BlockSpec((DH, D2), lambda i: (0, 0)),
            pl.BlockSpec((DH, D2), lambda i: (0, 0)),
            pl.BlockSpec((1, D2), lambda i: (0, 0)),
        ],
        out_specs=[
            pl.BlockSpec((BR, D2), lambda i: (i, 0)),
            pl.BlockSpec((BR, D2), lambda i: (i, 0)),
            pl.BlockSpec((BR, D2), lambda i: (i, 0)),
        ],
        out_shape=[
            jax.ShapeDtypeStruct((RN, D2), jnp.float32),
            jax.ShapeDtypeStruct((RN, D2), jnp.float32),
            jax.ShapeDtypeStruct((RN, D2), jnp.float32),
        ],
    )(p1, pdeg, xr, w2l, w2r, b2p)


def _tc3_body(p_ref, deg_ref, hr_ref, out_ref):
    pv = p_ref[...]
    out_ref[...] = (pv[0] + pv[1]) / deg_ref[...] + hr_ref[...]


def _tc3(p2, deg, hr):
    return pl.pallas_call(
        _tc3_body,
        grid=(GRID,),
        in_specs=[
            pl.BlockSpec((2, BR, D2), lambda i: (0, i, 0)),
            pl.BlockSpec((BR, D2), lambda i: (i, 0)),
            pl.BlockSpec((BR, D2), lambda i: (i, 0)),
        ],
        out_specs=pl.BlockSpec((BR, D2), lambda i: (i, 0)),
        out_shape=jax.ShapeDtypeStruct((RN, D2), jnp.float32),
    )(p2, deg, hr)


_MESH = plsc.VectorSubcoreMesh(core_axis_name="c", subcore_axis_name="s")
_RPT = RN // NTILE
NBUF = 4
GROUPS = CPT // NBUF


def _agg_loop(feat, srcb, dstb, rows, acc, gsem, per_chunk_extra):
    for b in range(NBUF):
        pltpu.async_copy(feat.at[srcb.at[b]], rows.at[b], gsem)
    plsc.subcore_barrier()

    def outer(gi, carry):
        for b in range(NBUF):
            i = gi * NBUF + b
            pltpu.make_async_copy(feat.at[srcb.at[i]], rows.at[b],
                                  gsem).wait()
            pltpu.sync_copy(rows.at[b], acc.at[dstb.at[i]], add=True)
            per_chunk_extra(i)

            @pl.when(i + NBUF < CPT)
            def _():
                pltpu.async_copy(feat.at[srcb.at[i + NBUF]], rows.at[b],
                                 gsem)
        return carry

    lax.fori_loop(0, GROUPS, outer, 0)


@functools.partial(
    pl.kernel,
    mesh=_MESH,
    compiler_params=pltpu.CompilerParams(use_tc_tiling_on_sc=False),
    out_type=[
        jax.ShapeDtypeStruct((NSC, RN, DH), jnp.float32),
        jax.ShapeDtypeStruct((NSC, RN, D2), jnp.float32),
    ],
    scratch_types=[
        pltpu.VMEM((CPT, CHUNK), jnp.int32),
        pltpu.VMEM((CPT, CHUNK), jnp.int32),
        pltpu.VMEM((NBUF, CHUNK, DH), jnp.float32),
        pltpu.VMEM((CHUNK, D2), jnp.float32),
        pltpu.VMEM_SHARED((RN, DH), jnp.float32),
        pltpu.VMEM_SHARED((RN, D2), jnp.float32),
        pltpu.SemaphoreType.DMA,
    ],
)
def _sc_agg1(feat, srcs, dsts, zeros64, zeros16, ones16,
             out, outdeg, srcb, dstb, rows, onesb, acc, accd, gsem):
    c = lax.axis_index("c")
    s = lax.axis_index("s")
    wid = s * NSC + c
    pltpu.sync_copy(zeros64.at[pl.ds(s * _RPT, _RPT)],
                    acc.at[pl.ds(s * _RPT, _RPT)])
    pltpu.sync_copy(zeros16.at[pl.ds(s * _RPT, _RPT)],
                    accd.at[pl.ds(s * _RPT, _RPT)])
    pltpu.sync_copy(ones16, onesb)
    pltpu.sync_copy(srcs.at[pl.ds(wid * CPT, CPT)], srcb)
    pltpu.sync_copy(dsts.at[pl.ds(wid * CPT, CPT)], dstb)

    def extra(i):
        pltpu.sync_copy(onesb, accd.at[dstb.at[i]], add=True)

    _agg_loop(feat, srcb, dstb, rows, acc, gsem, extra)
    plsc.subcore_barrier()
    pltpu.sync_copy(acc.at[pl.ds(s * _RPT, _RPT)],
                    out.at[c, pl.ds(s * _RPT, _RPT)])
    pltpu.sync_copy(accd.at[pl.ds(s * _RPT, _RPT)],
                    outdeg.at[c, pl.ds(s * _RPT, _RPT)])


@functools.partial(
    pl.kernel,
    mesh=_MESH,
    compiler_params=pltpu.CompilerParams(use_tc_tiling_on_sc=False),
    out_type=jax.ShapeDtypeStruct((NSC, RN, D2), jnp.float32),
    scratch_types=[
        pltpu.VMEM((CPT, CHUNK), jnp.int32),
        pltpu.VMEM((CPT, CHUNK), jnp.int32),
        pltpu.VMEM((NBUF, CHUNK, D2), jnp.float32),
        pltpu.VMEM_SHARED((RN, D2), jnp.float32),
        pltpu.SemaphoreType.DMA,
    ],
)
def _sc_agg2(feat, srcs, dsts, zeros16, out, srcb, dstb, rows, acc, gsem):
    c = lax.axis_index("c")
    s = lax.axis_index("s")
    wid = s * NSC + c
    pltpu.sync_copy(zeros16.at[pl.ds(s * _RPT, _RPT)],
                    acc.at[pl.ds(s * _RPT, _RPT)])
    pltpu.sync_copy(srcs.at[pl.ds(wid * CPT, CPT)], srcb)
    pltpu.sync_copy(dsts.at[pl.ds(wid * CPT, CPT)], dstb)

    _agg_loop(feat, srcb, dstb, rows, acc, gsem, lambda i: None)
    plsc.subcore_barrier()
    pltpu.sync_copy(acc.at[pl.ds(s * _RPT, _RPT)],
                    out.at[c, pl.ds(s * _RPT, _RPT)])


def kernel(x, edge_index, W1_l, W1_r, b1, W2_l, W2_r, b2):
    src = edge_index[0].astype(jnp.int32)
    dst = edge_index[1].astype(jnp.int32)
    srcp = jnp.zeros((EPAD,), jnp.int32).at[:E].set(src).reshape(NCH, CHUNK)
    junk = N + jnp.arange(EPAD, dtype=jnp.int32) % (RN - N)
    dstp = junk.at[:E].set(dst).reshape(NCH, CHUNK)
    xp = jnp.zeros((RN, DIN), jnp.float32).at[:N].set(x)

    xl, xr = _tc1(xp, W1_l, W1_r, b1[None, :])

    zeros64 = jnp.zeros((RN, DH), jnp.float32)
    zeros16 = jnp.zeros((RN, D2), jnp.float32)
    ones16 = jnp.ones((CHUNK, D2), jnp.float32)
    p1, pdeg = _sc_agg1(xl, srcp, dstp, zeros64, zeros16, ones16)

    w2l = jnp.zeros((DH, D2), jnp.float32).at[:, :DOUT].set(W2_l)
    w2r = jnp.zeros((DH, D2), jnp.float32).at[:, :DOUT].set(W2_r)
    b2p = jnp.zeros((1, D2), jnp.float32).at[0, :DOUT].set(b2)
    hl, hr, deg = _tc2(p1, pdeg, xr, w2l, w2r, b2p)

    p2 = _sc_agg2(hl, srcp, dstp, zeros16)

    outp = _tc3(p2, deg, hr)
    return outp[:N, :DOUT]


# speedup vs baseline: 15.9492x; 1.9904x over previous
import functools

import jax
import jax.numpy as jnp
from jax import lax
from jax.experimental import pallas as pl
from jax.experimental.pallas import tpu as pltpu
from jax.experimental.pallas import tpu_sc as plsc

N = 10000
E = 320000
DIN = 128
DH = 64
DOUT = 2

RN = N
D2 = 16
NSC = 2
NTILE = 16
NW = NSC * NTILE
CHUNK = 100
CPT = 100
NCH = NW * CPT

BR = 2000
GRID = RN // BR


def _tc1_body(x_ref, wl_ref, wr_ref, br_ref, xl_ref, xr_ref):
    xv = x_ref[...]
    xl_ref[...] = jnp.dot(xv, wl_ref[...], preferred_element_type=jnp.float32)
    xr_ref[...] = jnp.dot(xv, wr_ref[...],
                          preferred_element_type=jnp.float32) + br_ref[...]


def _tc1(xp, w1l, w1r, brow):
    return pl.pallas_call(
        _tc1_body,
        grid=(GRID,),
        in_specs=[
            pl.BlockSpec((BR, DIN), lambda i: (i, 0)),
            pl.BlockSpec((DIN, DH), lambda i: (0, 0)),
            pl.BlockSpec((DIN, DH), lambda i: (0, 0)),
            pl.BlockSpec((1, DH), lambda i: (0, 0)),
        ],
        out_specs=[
            pl.BlockSpec((BR, DH), lambda i: (i, 0)),
            pl.BlockSpec((BR, DH), lambda i: (i, 0)),
        ],
        out_shape=[
            jax.ShapeDtypeStruct((RN, DH), jnp.float32),
            jax.ShapeDtypeStruct((RN, DH), jnp.float32),
        ],
    )(xp, w1l, w1r, brow)


def _tc2_body(p_ref, pd_ref, xr_ref, wl_ref, wr_ref, b2_ref,
              hl_ref, hr_ref, deg_ref):
    pv = p_ref[...]
    pd = pd_ref[...]
    ssum = pv[0] + pv[1]
    deg = jnp.maximum(pd[0, :, 0:1] + pd[1, :, 0:1], 1.0)
    h = jnp.maximum(ssum / deg + xr_ref[...], 0.0)
    hl_ref[...] = jnp.dot(h, wl_ref[...], preferred_element_type=jnp.float32)
    hr_ref[...] = jnp.dot(h, wr_ref[...],
                          preferred_element_type=jnp.float32) + b2_ref[...]
    deg_ref[...] = jnp.broadcast_to(deg, (BR, D2))


def _tc2(p1, pdeg, xr, w2l, w2r, b2p):
    return pl.pallas_call(
        _tc2_body,
        grid=(GRID,),
        in_specs=[
            pl.BlockSpec((2, BR, DH), lambda i: (0, i, 0)),
            pl.BlockSpec((2, BR, D2), lambda i: (0, i, 0)),
            pl.BlockSpec((BR, DH), lambda i: (i, 0)),
            pl.BlockSpec((DH, D2), lambda i: (0, 0)),
            pl.BlockSpec((DH, D2), lambda i: (0, 0)),
            pl.BlockSpec((1, D2), lambda i: (0, 0)),
        ],
        out_specs=[
            pl.BlockSpec((BR, D2), lambda i: (i, 0)),
            pl.BlockSpec((BR, D2), lambda i: (i, 0)),
            pl.BlockSpec((BR, D2), lambda i: (i, 0)),
        ],
        out_shape=[
            jax.ShapeDtypeStruct((RN, D2), jnp.float32),
            jax.ShapeDtypeStruct((RN, D2), jnp.float32),
            jax.ShapeDtypeStruct((RN, D2), jnp.float32),
        ],
    )(p1, pdeg, xr, w2l, w2r, b2p)


def _tc3_body(p_ref, deg_ref, hr_ref, out_ref):
    pv = p_ref[...]
    out_ref[...] = (pv[0] + pv[1]) / deg_ref[...] + hr_ref[...]


def _tc3(p2, deg, hr):
    return pl.pallas_call(
        _tc3_body,
        grid=(GRID,),
        in_specs=[
            pl.BlockSpec((2, BR, D2), lambda i: (0, i, 0)),
            pl.BlockSpec((BR, D2), lambda i: (i, 0)),
            pl.BlockSpec((BR, D2), lambda i: (i, 0)),
        ],
        out_specs=pl.BlockSpec((BR, D2), lambda i: (i, 0)),
        out_shape=jax.ShapeDtypeStruct((RN, D2), jnp.float32),
    )(p2, deg, hr)


_MESH = plsc.VectorSubcoreMesh(core_axis_name="c", subcore_axis_name="s")
_RPT = RN // NTILE
NBUF = 4
GROUPS = CPT // NBUF


def _agg_loop(feat, srcb, dstb, rows, acc, gsem, per_chunk_extra):
    for b in range(NBUF):
        pltpu.async_copy(feat.at[srcb.at[b]], rows.at[b], gsem)
    plsc.subcore_barrier()

    def outer(gi, carry):
        for b in range(NBUF):
            i = gi * NBUF + b
            pltpu.make_async_copy(feat.at[srcb.at[i]], rows.at[b],
                                  gsem).wait()
            pltpu.sync_copy(rows.at[b], acc.at[dstb.at[i]], add=True)
            per_chunk_extra(i)

            @pl.when(i + NBUF < CPT)
            def _():
                pltpu.async_copy(feat.at[srcb.at[i + NBUF]], rows.at[b],
                                 gsem)
        return carry

    lax.fori_loop(0, GROUPS, outer, 0)


@functools.partial(
    pl.kernel,
    mesh=_MESH,
    compiler_params=pltpu.CompilerParams(use_tc_tiling_on_sc=False),
    out_type=[
        jax.ShapeDtypeStruct((NSC, RN, DH), jnp.float32),
        jax.ShapeDtypeStruct((NSC, RN, D2), jnp.float32),
    ],
    scratch_types=[
        pltpu.VMEM((CPT, CHUNK), jnp.int32),
        pltpu.VMEM((CPT, CHUNK), jnp.int32),
        pltpu.VMEM((NBUF, CHUNK, DH), jnp.float32),
        pltpu.VMEM((CHUNK, D2), jnp.float32),
        pltpu.VMEM_SHARED((RN, DH), jnp.float32),
        pltpu.VMEM_SHARED((RN, D2), jnp.float32),
        pltpu.SemaphoreType.DMA,
    ],
)
def _sc_agg1(feat, srcs, dsts, zeros64, zeros16, ones16,
             out, outdeg, srcb, dstb, rows, onesb, acc, accd, gsem):
    c = lax.axis_index("c")
    s = lax.axis_index("s")
    wid = s * NSC + c
    pltpu.sync_copy(zeros64.at[pl.ds(s * _RPT, _RPT)],
                    acc.at[pl.ds(s * _RPT, _RPT)])
    pltpu.sync_copy(zeros16.at[pl.ds(s * _RPT, _RPT)],
                    accd.at[pl.ds(s * _RPT, _RPT)])
    pltpu.sync_copy(ones16, onesb)
    pltpu.sync_copy(srcs.at[pl.ds(wid * CPT, CPT)], srcb)
    pltpu.sync_copy(dsts.at[pl.ds(wid * CPT, CPT)], dstb)

    def extra(i):
        pltpu.sync_copy(onesb, accd.at[dstb.at[i]], add=True)

    _agg_loop(feat, srcb, dstb, rows, acc, gsem, extra)
    plsc.subcore_barrier()
    pltpu.sync_copy(acc.at[pl.ds(s * _RPT, _RPT)],
                    out.at[c, pl.ds(s * _RPT, _RPT)])
    pltpu.sync_copy(accd.at[pl.ds(s * _RPT, _RPT)],
                    outdeg.at[c, pl.ds(s * _RPT, _RPT)])


@functools.partial(
    pl.kernel,
    mesh=_MESH,
    compiler_params=pltpu.CompilerParams(use_tc_tiling_on_sc=False),
    out_type=jax.ShapeDtypeStruct((NSC, RN, D2), jnp.float32),
    scratch_types=[
        pltpu.VMEM((CPT, CHUNK), jnp.int32),
        pltpu.VMEM((CPT, CHUNK), jnp.int32),
        pltpu.VMEM((NBUF, CHUNK, D2), jnp.float32),
        pltpu.VMEM_SHARED((RN, D2), jnp.float32),
        pltpu.SemaphoreType.DMA,
    ],
)
def _sc_agg2(feat, srcs, dsts, zeros16, out, srcb, dstb, rows, acc, gsem):
    c = lax.axis_index("c")
    s = lax.axis_index("s")
    wid = s * NSC + c
    pltpu.sync_copy(zeros16.at[pl.ds(s * _RPT, _RPT)],
                    acc.at[pl.ds(s * _RPT, _RPT)])
    pltpu.sync_copy(srcs.at[pl.ds(wid * CPT, CPT)], srcb)
    pltpu.sync_copy(dsts.at[pl.ds(wid * CPT, CPT)], dstb)

    _agg_loop(feat, srcb, dstb, rows, acc, gsem, lambda i: None)
    plsc.subcore_barrier()
    pltpu.sync_copy(acc.at[pl.ds(s * _RPT, _RPT)],
                    out.at[c, pl.ds(s * _RPT, _RPT)])


def kernel(x, edge_index, W1_l, W1_r, b1, W2_l, W2_r, b2):
    srcp = edge_index[0].astype(jnp.int32).reshape(NCH, CHUNK)
    dstp = edge_index[1].astype(jnp.int32).reshape(NCH, CHUNK)

    xl, xr = _tc1(x, W1_l, W1_r, b1[None, :])

    zeros64 = jnp.zeros((RN, DH), jnp.float32)
    zeros16 = jnp.zeros((RN, D2), jnp.float32)
    ones16 = jnp.ones((CHUNK, D2), jnp.float32)
    p1, pdeg = _sc_agg1(xl, srcp, dstp, zeros64, zeros16, ones16)

    w2l = jnp.zeros((DH, D2), jnp.float32).at[:, :DOUT].set(W2_l)
    w2r = jnp.zeros((DH, D2), jnp.float32).at[:, :DOUT].set(W2_r)
    b2p = jnp.zeros((1, D2), jnp.float32).at[0, :DOUT].set(b2)
    hl, hr, deg = _tc2(p1, pdeg, xr, w2l, w2r, b2p)

    p2 = _sc_agg2(hl, srcp, dstp, zeros16)

    outp = _tc3(p2, deg, hr)
    return outp[:N, :DOUT]


# speedup vs baseline: 16.4741x; 1.0329x over previous
import functools

import jax
import jax.numpy as jnp
from jax import lax
from jax.experimental import pallas as pl
from jax.experimental.pallas import tpu as pltpu
from jax.experimental.pallas import tpu_sc as plsc

N = 10000
E = 320000
DIN = 128
DH = 64
DOUT = 2

RN = N
D2 = 16
NSC = 2
NTILE = 16
NW = NSC * NTILE
CHUNK = 80
CPT = 125
EPT = CPT * CHUNK

BR = 2000
GRID = RN // BR


def _tc1_body(x_ref, wl_ref, wr_ref, br_ref, xl_ref, xr_ref):
    xv = x_ref[...]
    xl_ref[...] = jnp.dot(xv, wl_ref[...], preferred_element_type=jnp.float32)
    xr_ref[...] = jnp.dot(xv, wr_ref[...],
                          preferred_element_type=jnp.float32) + br_ref[...]


def _tc1(xp, w1l, w1r, brow):
    return pl.pallas_call(
        _tc1_body,
        grid=(GRID,),
        in_specs=[
            pl.BlockSpec((BR, DIN), lambda i: (i, 0)),
            pl.BlockSpec((DIN, DH), lambda i: (0, 0)),
            pl.BlockSpec((DIN, DH), lambda i: (0, 0)),
            pl.BlockSpec((1, DH), lambda i: (0, 0)),
        ],
        out_specs=[
            pl.BlockSpec((BR, DH), lambda i: (i, 0)),
            pl.BlockSpec((BR, DH), lambda i: (i, 0)),
        ],
        out_shape=[
            jax.ShapeDtypeStruct((RN, DH), jnp.float32),
            jax.ShapeDtypeStruct((RN, DH), jnp.float32),
        ],
    )(xp, w1l, w1r, brow)


def _tc2_body(p_ref, pd_ref, xr_ref, wl_ref, wr_ref, b2_ref,
              hl_ref, hr_ref, deg_ref):
    pv = p_ref[...]
    pd = pd_ref[...]
    ssum = pv[0] + pv[1]
    deg = jnp.maximum(pd[0, :, 0:1] + pd[1, :, 0:1], 1.0)
    h = jnp.maximum(ssum / deg + xr_ref[...], 0.0)
    hl_ref[...] = jnp.dot(h, wl_ref[...], preferred_element_type=jnp.float32)
    hr_ref[...] = jnp.dot(h, wr_ref[...],
                          preferred_element_type=jnp.float32) + b2_ref[...]
    deg_ref[...] = jnp.broadcast_to(deg, (BR, D2))


def _tc2(p1, pdeg, xr, w2l, w2r, b2p):
    return pl.pallas_call(
        _tc2_body,
        grid=(GRID,),
        in_specs=[
            pl.BlockSpec((2, BR, DH), lambda i: (0, i, 0)),
            pl.BlockSpec((2, BR, D2), lambda i: (0, i, 0)),
            pl.BlockSpec((BR, DH), lambda i: (i, 0)),
            pl.BlockSpec((DH, D2), lambda i: (0, 0)),
            pl.BlockSpec((DH, D2), lambda i: (0, 0)),
            pl.BlockSpec((1, D2), lambda i: (0, 0)),
        ],
        out_specs=[
            pl.BlockSpec((BR, D2), lambda i: (i, 0)),
            pl.BlockSpec((BR, D2), lambda i: (i, 0)),
            pl.BlockSpec((BR, D2), lambda i: (i, 0)),
        ],
        out_shape=[
            jax.ShapeDtypeStruct((RN, D2), jnp.float32),
            jax.ShapeDtypeStruct((RN, D2), jnp.float32),
            jax.ShapeDtypeStruct((RN, D2), jnp.float32),
        ],
    )(p1, pdeg, xr, w2l, w2r, b2p)


def _tc3_body(p_ref, deg_ref, hr_ref, out_ref):
    pv = p_ref[...]
    out_ref[...] = (pv[0] + pv[1]) / deg_ref[...] + hr_ref[...]


def _tc3(p2, deg, hr):
    return pl.pallas_call(
        _tc3_body,
        grid=(GRID,),
        in_specs=[
            pl.BlockSpec((2, BR, D2), lambda i: (0, i, 0)),
            pl.BlockSpec((BR, D2), lambda i: (i, 0)),
            pl.BlockSpec((BR, D2), lambda i: (i, 0)),
        ],
        out_specs=pl.BlockSpec((BR, D2), lambda i: (i, 0)),
        out_shape=jax.ShapeDtypeStruct((RN, D2), jnp.float32),
    )(p2, deg, hr)


_MESH = plsc.VectorSubcoreMesh(core_axis_name="c", subcore_axis_name="s")
_RPT = RN // NTILE
NBUF = 5
GROUPS = CPT // NBUF


def _idx(buf, i):
    return buf.at[pl.ds(pl.multiple_of(i * CHUNK, 8), CHUNK)]


def _agg_loop(feat, srcb, dstb, rows, acc, gsem, per_chunk_extra):
    for b in range(NBUF):
        pltpu.async_copy(feat.at[_idx(srcb, b)], rows.at[b], gsem)
    plsc.subcore_barrier()

    def outer(gi, carry):
        for b in range(NBUF):
            i = gi * NBUF + b
            pltpu.make_async_copy(feat.at[_idx(srcb, i)], rows.at[b],
                                  gsem).wait()
            pltpu.sync_copy(rows.at[b], acc.at[_idx(dstb, i)], add=True)
            per_chunk_extra(i)

            @pl.when(i + NBUF < CPT)
            def _():
                pltpu.async_copy(feat.at[_idx(srcb, i + NBUF)], rows.at[b],
                                 gsem)
        return carry

    lax.fori_loop(0, GROUPS, outer, 0)


@functools.partial(
    pl.kernel,
    mesh=_MESH,
    compiler_params=pltpu.CompilerParams(use_tc_tiling_on_sc=False),
    out_type=[
        jax.ShapeDtypeStruct((NSC, RN, DH), jnp.float32),
        jax.ShapeDtypeStruct((NSC, RN, D2), jnp.float32),
    ],
    scratch_types=[
        pltpu.VMEM((EPT,), jnp.int32),
        pltpu.VMEM((EPT,), jnp.int32),
        pltpu.VMEM((NBUF, CHUNK, DH), jnp.float32),
        pltpu.VMEM((CHUNK, D2), jnp.float32),
        pltpu.VMEM_SHARED((RN, DH), jnp.float32),
        pltpu.VMEM_SHARED((RN, D2), jnp.float32),
        pltpu.SemaphoreType.DMA,
    ],
)
def _sc_agg1(feat, srcs, dsts, zeros64, zeros16, ones16,
             out, outdeg, srcb, dstb, rows, onesb, acc, accd, gsem):
    c = lax.axis_index("c")
    s = lax.axis_index("s")
    wid = s * NSC + c
    pltpu.sync_copy(zeros64.at[pl.ds(s * _RPT, _RPT)],
                    acc.at[pl.ds(s * _RPT, _RPT)])
    pltpu.sync_copy(zeros16.at[pl.ds(s * _RPT, _RPT)],
                    accd.at[pl.ds(s * _RPT, _RPT)])
    pltpu.sync_copy(ones16, onesb)
    pltpu.sync_copy(srcs.at[pl.ds(wid * EPT, EPT)], srcb)
    pltpu.sync_copy(dsts.at[pl.ds(wid * EPT, EPT)], dstb)

    def extra(i):
        pltpu.sync_copy(onesb, accd.at[_idx(dstb, i)], add=True)

    _agg_loop(feat, srcb, dstb, rows, acc, gsem, extra)
    plsc.subcore_barrier()
    pltpu.sync_copy(acc.at[pl.ds(s * _RPT, _RPT)],
                    out.at[c, pl.ds(s * _RPT, _RPT)])
    pltpu.sync_copy(accd.at[pl.ds(s * _RPT, _RPT)],
                    outdeg.at[c, pl.ds(s * _RPT, _RPT)])


@functools.partial(
    pl.kernel,
    mesh=_MESH,
    compiler_params=pltpu.CompilerParams(use_tc_tiling_on_sc=False),
    out_type=jax.ShapeDtypeStruct((NSC, RN, D2), jnp.float32),
    scratch_types=[
        pltpu.VMEM((EPT,), jnp.int32),
        pltpu.VMEM((EPT,), jnp.int32),
        pltpu.VMEM((NBUF, CHUNK, D2), jnp.float32),
        pltpu.VMEM_SHARED((RN, D2), jnp.float32),
        pltpu.SemaphoreType.DMA,
    ],
)
def _sc_agg2(feat, srcs, dsts, zeros16, out, srcb, dstb, rows, acc, gsem):
    c = lax.axis_index("c")
    s = lax.axis_index("s")
    wid = s * NSC + c
    pltpu.sync_copy(zeros16.at[pl.ds(s * _RPT, _RPT)],
                    acc.at[pl.ds(s * _RPT, _RPT)])
    pltpu.sync_copy(srcs.at[pl.ds(wid * EPT, EPT)], srcb)
    pltpu.sync_copy(dsts.at[pl.ds(wid * EPT, EPT)], dstb)

    _agg_loop(feat, srcb, dstb, rows, acc, gsem, lambda i: None)
    plsc.subcore_barrier()
    pltpu.sync_copy(acc.at[pl.ds(s * _RPT, _RPT)],
                    out.at[c, pl.ds(s * _RPT, _RPT)])


def kernel(x, edge_index, W1_l, W1_r, b1, W2_l, W2_r, b2):
    srcp = edge_index[0].astype(jnp.int32)
    dstp = edge_index[1].astype(jnp.int32)

    xl, xr = _tc1(x, W1_l, W1_r, b1[None, :])

    zeros64 = jnp.zeros((RN, DH), jnp.float32)
    zeros16 = jnp.zeros((RN, D2), jnp.float32)
    ones16 = jnp.ones((CHUNK, D2), jnp.float32)
    p1, pdeg = _sc_agg1(xl, srcp, dstp, zeros64, zeros16, ones16)

    w2l = jnp.zeros((DH, D2), jnp.float32).at[:, :DOUT].set(W2_l)
    w2r = jnp.zeros((DH, D2), jnp.float32).at[:, :DOUT].set(W2_r)
    b2p = jnp.zeros((1, D2), jnp.float32).at[0, :DOUT].set(b2)
    hl, hr, deg = _tc2(p1, pdeg, xr, w2l, w2r, b2p)

    p2 = _sc_agg2(hl, srcp, dstp, zeros16)

    outp = _tc3(p2, deg, hr)
    return outp[:N, :DOUT]


# speedup vs baseline: 18.3514x; 1.1139x over previous
import functools

import jax
import jax.numpy as jnp
from jax import lax
from jax.experimental import pallas as pl
from jax.experimental.pallas import tpu as pltpu
from jax.experimental.pallas import tpu_sc as plsc

N = 10000
E = 320000
DIN = 128
DH = 64
DOUT = 2

RN = N
RND = 10240
NSC = 2
NTILE = 16
NW = NSC * NTILE
CHUNK = 80
CPT = 125
EPT = CPT * CHUNK
EPT2 = E // NTILE
SCR = RND // NSC
TPR = SCR // NTILE

BR = 2000
GRID = RN // BR


def _tc1_body(x_ref, wl_ref, wr_ref, br_ref, xl_ref, xr_ref):
    xv = x_ref[...]
    xl_ref[...] = jnp.dot(xv, wl_ref[...], preferred_element_type=jnp.float32)
    xr_ref[...] = jnp.dot(xv, wr_ref[...],
                          preferred_element_type=jnp.float32) + br_ref[...]


def _tc1(xp, w1l, w1r, brow):
    return pl.pallas_call(
        _tc1_body,
        grid=(GRID,),
        in_specs=[
            pl.BlockSpec((BR, DIN), lambda i: (i, 0)),
            pl.BlockSpec((DIN, DH), lambda i: (0, 0)),
            pl.BlockSpec((DIN, DH), lambda i: (0, 0)),
            pl.BlockSpec((1, DH), lambda i: (0, 0)),
        ],
        out_specs=[
            pl.BlockSpec((BR, DH), lambda i: (i, 0)),
            pl.BlockSpec((BR, DH), lambda i: (i, 0)),
        ],
        out_shape=[
            jax.ShapeDtypeStruct((RN, DH), jnp.float32),
            jax.ShapeDtypeStruct((RN, DH), jnp.float32),
        ],
    )(xp, w1l, w1r, brow)


def _tc2_body(p_ref, deg_ref, xr_ref, wl_ref, wr_ref, b2_ref,
              hl_ref, hr_ref, degout_ref):
    pv = p_ref[...]
    ssum = pv[0] + pv[1]
    deg = jnp.maximum(deg_ref[...], 1.0)
    h = jnp.maximum(ssum / deg + xr_ref[...], 0.0)
    hl_ref[...] = jnp.dot(h, wl_ref[...], preferred_element_type=jnp.float32)
    hr_ref[...] = jnp.dot(h, wr_ref[...],
                          preferred_element_type=jnp.float32) + b2_ref[...]
    degout_ref[...] = jnp.broadcast_to(deg, (BR, DOUT))


def _tc2(p1, degsum, xr, w2l, w2r, b2p):
    return pl.pallas_call(
        _tc2_body,
        grid=(GRID,),
        in_specs=[
            pl.BlockSpec((2, BR, DH), lambda i: (0, i, 0)),
            pl.BlockSpec((BR, 1), lambda i: (i, 0)),
            pl.BlockSpec((BR, DH), lambda i: (i, 0)),
            pl.BlockSpec((DH, DOUT), lambda i: (0, 0)),
            pl.BlockSpec((DH, DOUT), lambda i: (0, 0)),
            pl.BlockSpec((1, DOUT), lambda i: (0, 0)),
        ],
        out_specs=[
            pl.BlockSpec((BR, DOUT), lambda i: (i, 0)),
            pl.BlockSpec((BR, DOUT), lambda i: (i, 0)),
            pl.BlockSpec((BR, DOUT), lambda i: (i, 0)),
        ],
        out_shape=[
            jax.ShapeDtypeStruct((RN, DOUT), jnp.float32),
            jax.ShapeDtypeStruct((RN, DOUT), jnp.float32),
            jax.ShapeDtypeStruct((RN, DOUT), jnp.float32),
        ],
    )(p1, degsum, xr, w2l, w2r, b2p)


_MESH = plsc.VectorSubcoreMesh(core_axis_name="c", subcore_axis_name="s")
_RPT = RN // NTILE
DRPT = RND // NTILE
NBUF = 5
GROUPS = CPT // NBUF


def _idx(buf, i):
    return buf.at[pl.ds(pl.multiple_of(i * CHUNK, 8), CHUNK)]


@functools.partial(
    pl.kernel,
    mesh=_MESH,
    compiler_params=pltpu.CompilerParams(use_tc_tiling_on_sc=False,
                                        needs_layout_passes=False),
    out_type=[
        jax.ShapeDtypeStruct((NSC, RN, DH), jnp.float32),
        jax.ShapeDtypeStruct((NSC, RND), jnp.float32),
    ],
    scratch_types=[
        pltpu.VMEM((EPT,), jnp.int32),
        pltpu.VMEM((EPT,), jnp.int32),
        pltpu.VMEM((NBUF, CHUNK, DH), jnp.float32),
        pltpu.VMEM((RND,), jnp.float32),
        pltpu.VMEM((NTILE, DRPT), jnp.float32),
        pltpu.VMEM((DRPT,), jnp.float32),
        pltpu.VMEM_SHARED((RN, DH), jnp.float32),
        pltpu.VMEM_SHARED((NTILE, RND), jnp.float32),
        pltpu.SemaphoreType.DMA,
        pltpu.SemaphoreType.DMA,
    ],
)
def _sc_agg1(feat, srcs, dsts, zeros64, out, outdeg,
             srcb, dstb, rows, degv, rbuf, dout, acc, dstage, gsem, ssem):
    c = lax.axis_index("c")
    s = lax.axis_index("s")
    wid = s * NSC + c
    pltpu.sync_copy(zeros64.at[pl.ds(s * _RPT, _RPT)],
                    acc.at[pl.ds(s * _RPT, _RPT)])
    pltpu.sync_copy(srcs.at[pl.ds(wid * EPT, EPT)], srcb)
    pltpu.sync_copy(dsts.at[pl.ds(wid * EPT, EPT)], dstb)
    z16 = jnp.zeros((16,), jnp.float32)

    def zbody(k, carry):
        degv[pl.ds(pl.multiple_of(k * 16, 16), 16)] = z16
        return carry

    lax.fori_loop(0, RND // 16, zbody, 0)
    ones_v = jnp.ones((16,), jnp.float32)

    for b in range(NBUF):
        pltpu.async_copy(feat.at[_idx(srcb, b)], rows.at[b], gsem)
    plsc.subcore_barrier()

    def outer(gi, carry):
        for b in range(NBUF):
            i = gi * NBUF + b
            pltpu.make_async_copy(feat.at[_idx(srcb, i)], rows.at[b],
                                  gsem).wait()
            d = pltpu.async_copy(rows.at[b], acc.at[_idx(dstb, i)], ssem,
                                 add=True)
            for k in range(CHUNK // 16):
                off = pl.multiple_of(i * CHUNK + k * 16, 16)
                dv = dstb[pl.ds(off, 16)]
                plsc.addupdate_scatter(degv, [dv], ones_v)
            d.wait()

            @pl.when(i + NBUF < CPT)
            def _():
                pltpu.async_copy(feat.at[_idx(srcb, i + NBUF)], rows.at[b],
                                 gsem)
        return carry

    lax.fori_loop(0, GROUPS, outer, 0)
    pltpu.sync_copy(degv, dstage.at[s])
    plsc.subcore_barrier()
    pltpu.sync_copy(acc.at[pl.ds(s * _RPT, _RPT)],
                    out.at[c, pl.ds(s * _RPT, _RPT)])
    pltpu.sync_copy(dstage.at[:, pl.ds(s * DRPT, DRPT)], rbuf)

    def rbody(k, carry):
        o = pl.multiple_of(k * 16, 16)
        v = rbuf[0, pl.ds(o, 16)]
        for j in range(1, NTILE):
            v = v + rbuf[j, pl.ds(o, 16)]
        dout[pl.ds(o, 16)] = v
        return carry

    lax.fori_loop(0, DRPT // 16, rbody, 0)
    pltpu.sync_copy(dout, outdeg.at[c, pl.ds(s * DRPT, DRPT)])


@functools.partial(
    pl.kernel,
    mesh=_MESH,
    compiler_params=pltpu.CompilerParams(use_tc_tiling_on_sc=False,
                                        needs_layout_passes=False),
    out_type=jax.ShapeDtypeStruct((RND * DOUT,), jnp.float32),
    scratch_types=[
        pltpu.VMEM((EPT2,), jnp.int32),
        pltpu.VMEM((EPT2,), jnp.int32),
        pltpu.VMEM((N * DOUT,), jnp.float32),
        pltpu.VMEM((SCR * DOUT,), jnp.float32),
        pltpu.VMEM((NTILE, TPR * DOUT), jnp.float32),
        pltpu.VMEM((TPR * DOUT,), jnp.float32),
        pltpu.VMEM((TPR * DOUT,), jnp.float32),
        pltpu.VMEM((TPR * DOUT,), jnp.float32),
        pltpu.VMEM_SHARED((NTILE, SCR * DOUT), jnp.float32),
    ],
)
def _sc_l2(hl, srcs, dsts, deg, hr, out,
           srcb, dstb, hlb, acc2, rbuf, hrb, degb, dout, sstage):
    c = lax.axis_index("c")
    s = lax.axis_index("s")
    lo = c * SCR
    base = c * SCR * DOUT + s * TPR * DOUT
    pltpu.sync_copy(srcs.at[pl.ds(s * EPT2, EPT2)], srcb)
    pltpu.sync_copy(dsts.at[pl.ds(s * EPT2, EPT2)], dstb)
    pltpu.sync_copy(hl, hlb)
    pltpu.sync_copy(deg.at[pl.ds(base, TPR * DOUT)], degb)
    pltpu.sync_copy(hr.at[pl.ds(base, TPR * DOUT)], hrb)
    z16 = jnp.zeros((16,), jnp.float32)

    def zbody(k, carry):
        acc2[pl.ds(pl.multiple_of(k * 16, 16), 16)] = z16
        return carry

    lax.fori_loop(0, SCR * DOUT // 16, zbody, 0)

    def body(k, carry):
        off = pl.multiple_of(k * 16, 16)
        sv = srcb[pl.ds(off, 16)]
        dv = dstb[pl.ds(off, 16)]
        m = jnp.logical_and(dv >= lo, dv < lo + SCR)
        dl = jnp.where(m, dv - lo, 0) * 2
        s2 = sv * 2
        g0 = plsc.load_gather(hlb, [s2])
        g1 = plsc.load_gather(hlb, [s2 + 1])
        plsc.addupdate_scatter(acc2, [dl], g0, mask=m)
        plsc.addupdate_scatter(acc2, [dl + 1], g1, mask=m)
        return carry

    lax.fori_loop(0, EPT2 // 16, body, 0)
    pltpu.sync_copy(acc2, sstage.at[s])
    plsc.subcore_barrier()
    pltpu.sync_copy(sstage.at[:, pl.ds(s * TPR * DOUT, TPR * DOUT)], rbuf)

    def rbody(k, carry):
        o = pl.multiple_of(k * 16, 16)
        v = rbuf[0, pl.ds(o, 16)]
        for j in range(1, NTILE):
            v = v + rbuf[j, pl.ds(o, 16)]
        dout[pl.ds(o, 16)] = v / degb[pl.ds(o, 16)] + hrb[pl.ds(o, 16)]
        return carry

    lax.fori_loop(0, TPR * DOUT // 16, rbody, 0)
    pltpu.sync_copy(dout, out.at[pl.ds(base, TPR * DOUT)])


def kernel(x, edge_index, W1_l, W1_r, b1, W2_l, W2_r, b2):
    srcp = edge_index[0].astype(jnp.int32)
    dstp = edge_index[1].astype(jnp.int32)

    xl, xr = _tc1(x, W1_l, W1_r, b1[None, :])

    zeros64 = jnp.zeros((RN, DH), jnp.float32)
    p1, pdeg = _sc_agg1(xl, srcp, dstp, zeros64)

    degsum = (pdeg[0, :RN] + pdeg[1, :RN])[:, None]
    hl, hr, deg = _tc2(p1, degsum, xr, W2_l, W2_r, b2[None, :])

    hlf = hl.reshape(N * DOUT)
    degf = jnp.pad(deg, ((0, RND - N), (0, 0)),
                   constant_values=1.0).reshape(RND * DOUT)
    hrf = jnp.pad(hr, ((0, RND - N), (0, 0))).reshape(RND * DOUT)
    outf = _sc_l2(hlf, srcp, dstp, degf, hrf)
    return outf.reshape(RND, DOUT)[:N]


# speedup vs baseline: 20.0384x; 1.0919x over previous
import functools

import jax
import jax.numpy as jnp
from jax import lax
from jax.experimental import pallas as pl
from jax.experimental.pallas import tpu as pltpu
from jax.experimental.pallas import tpu_sc as plsc

N = 10000
E = 320000
DIN = 128
DH = 64
DOUT = 2

RN = N
RND = 10240
NSC = 2
NTILE = 16
NW = NSC * NTILE
CHUNK = 80
CPT = 125
EPT = CPT * CHUNK
EPT2 = E // NTILE
SCR = RND // NSC
TPR = SCR // NTILE

BR = 2000
GRID = RN // BR


def _tc1_body(x_ref, wl_ref, wr_ref, br_ref, xl_ref, xr_ref):
    xv = x_ref[...]
    xl_ref[...] = jnp.dot(xv, wl_ref[...], preferred_element_type=jnp.float32)
    xr_ref[...] = jnp.dot(xv, wr_ref[...],
                          preferred_element_type=jnp.float32) + br_ref[...]


def _tc1(xp, w1l, w1r, brow):
    return pl.pallas_call(
        _tc1_body,
        grid=(GRID,),
        in_specs=[
            pl.BlockSpec((BR, DIN), lambda i: (i, 0)),
            pl.BlockSpec((DIN, DH), lambda i: (0, 0)),
            pl.BlockSpec((DIN, DH), lambda i: (0, 0)),
            pl.BlockSpec((1, DH), lambda i: (0, 0)),
        ],
        out_specs=[
            pl.BlockSpec((BR, DH), lambda i: (i, 0)),
            pl.BlockSpec((BR, DH), lambda i: (i, 0)),
        ],
        out_shape=[
            jax.ShapeDtypeStruct((RN, DH), jnp.float32),
            jax.ShapeDtypeStruct((RN, DH), jnp.float32),
        ],
    )(xp, w1l, w1r, brow)


def _tc2_body(p_ref, deg_ref, xr_ref, wl_ref, wr_ref, b2_ref,
              hl_ref, hr_ref):
    pv = p_ref[...]
    ssum = pv[0] + pv[1]
    deg = jnp.maximum(deg_ref[...], 1.0)
    h = jnp.maximum(ssum / deg + xr_ref[...], 0.0)
    hl_ref[...] = jnp.dot(h, wl_ref[...], preferred_element_type=jnp.float32)
    hr_ref[...] = jnp.dot(h, wr_ref[...],
                          preferred_element_type=jnp.float32) + b2_ref[...]


def _tc2(p1, degsum, xr, w2l, w2r, b2p):
    return pl.pallas_call(
        _tc2_body,
        grid=(GRID,),
        in_specs=[
            pl.BlockSpec((2, BR, DH), lambda i: (0, i, 0)),
            pl.BlockSpec((BR, 1), lambda i: (i, 0)),
            pl.BlockSpec((BR, DH), lambda i: (i, 0)),
            pl.BlockSpec((DH, DOUT), lambda i: (0, 0)),
            pl.BlockSpec((DH, DOUT), lambda i: (0, 0)),
            pl.BlockSpec((1, DOUT), lambda i: (0, 0)),
        ],
        out_specs=[
            pl.BlockSpec((BR, DOUT), lambda i: (i, 0)),
            pl.BlockSpec((BR, DOUT), lambda i: (i, 0)),
        ],
        out_shape=[
            jax.ShapeDtypeStruct((RN, DOUT), jnp.float32),
            jax.ShapeDtypeStruct((RN, DOUT), jnp.float32),
        ],
    )(p1, degsum, xr, w2l, w2r, b2p)


_MESH = plsc.VectorSubcoreMesh(core_axis_name="c", subcore_axis_name="s")
_RPT = RN // NTILE
DRPT = RND // NTILE
NBUF = 5
GROUPS = CPT // NBUF


def _idx(buf, i):
    return buf.at[pl.ds(pl.multiple_of(i * CHUNK, 8), CHUNK)]


@functools.partial(
    pl.kernel,
    mesh=_MESH,
    compiler_params=pltpu.CompilerParams(use_tc_tiling_on_sc=False,
                                        needs_layout_passes=False),
    out_type=[
        jax.ShapeDtypeStruct((NSC, RN, DH), jnp.float32),
        jax.ShapeDtypeStruct((NSC, RND), jnp.float32),
    ],
    scratch_types=[
        pltpu.VMEM((EPT,), jnp.int32),
        pltpu.VMEM((EPT,), jnp.int32),
        pltpu.VMEM((NBUF, CHUNK, DH), jnp.float32),
        pltpu.VMEM((RND,), jnp.float32),
        pltpu.VMEM((NTILE, DRPT), jnp.float32),
        pltpu.VMEM((DRPT,), jnp.float32),
        pltpu.VMEM_SHARED((RN, DH), jnp.float32),
        pltpu.VMEM_SHARED((NTILE, RND), jnp.float32),
        pltpu.SemaphoreType.DMA,
        pltpu.SemaphoreType.DMA,
    ],
)
def _sc_agg1(feat, srcs, dsts, zeros64, out, outdeg,
             srcb, dstb, rows, degv, rbuf, dout, acc, dstage, gsem, ssem):
    c = lax.axis_index("c")
    s = lax.axis_index("s")
    wid = s * NSC + c
    pltpu.sync_copy(zeros64.at[pl.ds(s * _RPT, _RPT)],
                    acc.at[pl.ds(s * _RPT, _RPT)])
    pltpu.sync_copy(srcs.at[pl.ds(wid * EPT, EPT)], srcb)
    pltpu.sync_copy(dsts.at[pl.ds(wid * EPT, EPT)], dstb)
    z16 = jnp.zeros((16,), jnp.float32)

    def zbody(k, carry):
        degv[pl.ds(pl.multiple_of(k * 16, 16), 16)] = z16
        return carry

    lax.fori_loop(0, RND // 16, zbody, 0)
    ones_v = jnp.ones((16,), jnp.float32)

    for b in range(NBUF):
        pltpu.async_copy(feat.at[_idx(srcb, b)], rows.at[b], gsem)
    plsc.subcore_barrier()

    def outer(gi, carry):
        for b in range(NBUF):
            i = gi * NBUF + b
            pltpu.make_async_copy(feat.at[_idx(srcb, i)], rows.at[b],
                                  gsem).wait()
            d = pltpu.async_copy(rows.at[b], acc.at[_idx(dstb, i)], ssem,
                                 add=True)
            for k in range(CHUNK // 16):
                off = pl.multiple_of(i * CHUNK + k * 16, 16)
                dv = dstb[pl.ds(off, 16)]
                plsc.addupdate_scatter(degv, [dv], ones_v)
            d.wait()

            @pl.when(i + NBUF < CPT)
            def _():
                pltpu.async_copy(feat.at[_idx(srcb, i + NBUF)], rows.at[b],
                                 gsem)
        return carry

    lax.fori_loop(0, GROUPS, outer, 0)
    pltpu.sync_copy(degv, dstage.at[s])
    plsc.subcore_barrier()
    pltpu.sync_copy(acc.at[pl.ds(s * _RPT, _RPT)],
                    out.at[c, pl.ds(s * _RPT, _RPT)])
    pltpu.sync_copy(dstage.at[:, pl.ds(s * DRPT, DRPT)], rbuf)

    def rbody(k, carry):
        o = pl.multiple_of(k * 16, 16)
        v = rbuf[0, pl.ds(o, 16)]
        for j in range(1, NTILE):
            v = v + rbuf[j, pl.ds(o, 16)]
        dout[pl.ds(o, 16)] = v
        return carry

    lax.fori_loop(0, DRPT // 16, rbody, 0)
    pltpu.sync_copy(dout, outdeg.at[c, pl.ds(s * DRPT, DRPT)])


@functools.partial(
    pl.kernel,
    mesh=_MESH,
    compiler_params=pltpu.CompilerParams(use_tc_tiling_on_sc=False,
                                        needs_layout_passes=False),
    out_type=[
        jax.ShapeDtypeStruct((RND,), jnp.float32),
        jax.ShapeDtypeStruct((RND,), jnp.float32),
    ],
    scratch_types=[
        pltpu.VMEM((EPT2,), jnp.int32),
        pltpu.VMEM((EPT2,), jnp.int32),
        pltpu.VMEM((RND,), jnp.float32),
        pltpu.VMEM((RND,), jnp.float32),
        pltpu.VMEM((SCR,), jnp.float32),
        pltpu.VMEM((SCR,), jnp.float32),
        pltpu.VMEM((NTILE, TPR), jnp.float32),
        pltpu.VMEM((NTILE, TPR), jnp.float32),
        pltpu.VMEM((TPR,), jnp.float32),
        pltpu.VMEM((TPR,), jnp.float32),
        pltpu.VMEM((TPR,), jnp.float32),
        pltpu.VMEM((TPR,), jnp.float32),
        pltpu.VMEM((TPR,), jnp.float32),
        pltpu.VMEM_SHARED((NTILE, 2, SCR), jnp.float32),
    ],
)
def _sc_l2(hl0, hl1, srcs, dsts, deg, hr0, hr1, out0, out1,
           srcb, dstb, hl0b, hl1b, acc0, acc1, rbuf0, rbuf1,
           hr0b, hr1b, degb, dout0, dout1, sstage):
    c = lax.axis_index("c")
    s = lax.axis_index("s")
    lo = c * SCR
    base = c * SCR + s * TPR
    pltpu.sync_copy(srcs.at[pl.ds(s * EPT2, EPT2)], srcb)
    pltpu.sync_copy(dsts.at[pl.ds(s * EPT2, EPT2)], dstb)
    pltpu.sync_copy(hl0, hl0b)
    pltpu.sync_copy(hl1, hl1b)
    pltpu.sync_copy(deg.at[pl.ds(base, TPR)], degb)
    pltpu.sync_copy(hr0.at[pl.ds(base, TPR)], hr0b)
    pltpu.sync_copy(hr1.at[pl.ds(base, TPR)], hr1b)
    z16 = jnp.zeros((16,), jnp.float32)

    def zbody(k, carry):
        o = pl.multiple_of(k * 16, 16)
        acc0[pl.ds(o, 16)] = z16
        acc1[pl.ds(o, 16)] = z16
        return carry

    lax.fori_loop(0, SCR // 16, zbody, 0)

    def body(k, carry):
        off = pl.multiple_of(k * 16, 16)
        sv = srcb[pl.ds(off, 16)]
        dv = dstb[pl.ds(off, 16)]
        m = jnp.logical_and(dv >= lo, dv < lo + SCR)
        dl = jnp.where(m, dv - lo, 0)
        g0 = plsc.load_gather(hl0b, [sv])
        g1 = plsc.load_gather(hl1b, [sv])
        plsc.addupdate_scatter(acc0, [dl], g0, mask=m)
        plsc.addupdate_scatter(acc1, [dl], g1, mask=m)
        return carry

    lax.fori_loop(0, EPT2 // 16, body, 0)
    pltpu.sync_copy(acc0, sstage.at[s, 0])
    pltpu.sync_copy(acc1, sstage.at[s, 1])
    plsc.subcore_barrier()
    pltpu.sync_copy(sstage.at[:, 0, pl.ds(s * TPR, TPR)], rbuf0)
    pltpu.sync_copy(sstage.at[:, 1, pl.ds(s * TPR, TPR)], rbuf1)

    def rbody(k, carry):
        o = pl.multiple_of(k * 16, 16)
        v0 = rbuf0[0, pl.ds(o, 16)]
        v1 = rbuf1[0, pl.ds(o, 16)]
        for j in range(1, NTILE):
            v0 = v0 + rbuf0[j, pl.ds(o, 16)]
            v1 = v1 + rbuf1[j, pl.ds(o, 16)]
        d = degb[pl.ds(o, 16)]
        dout0[pl.ds(o, 16)] = v0 / d + hr0b[pl.ds(o, 16)]
        dout1[pl.ds(o, 16)] = v1 / d + hr1b[pl.ds(o, 16)]
        return carry

    lax.fori_loop(0, TPR // 16, rbody, 0)
    pltpu.sync_copy(dout0, out0.at[pl.ds(base, TPR)])
    pltpu.sync_copy(dout1, out1.at[pl.ds(base, TPR)])


def kernel(x, edge_index, W1_l, W1_r, b1, W2_l, W2_r, b2):
    srcp = edge_index[0].astype(jnp.int32)
    dstp = edge_index[1].astype(jnp.int32)

    xl, xr = _tc1(x, W1_l, W1_r, b1[None, :])

    zeros64 = jnp.zeros((RN, DH), jnp.float32)
    p1, pdeg = _sc_agg1(xl, srcp, dstp, zeros64)

    degfull = pdeg[0] + pdeg[1]
    hl, hr = _tc2(p1, degfull[:RN, None], xr, W2_l, W2_r, b2[None, :])

    pad = (0, RND - N)
    hl0 = jnp.pad(hl[:, 0], pad)
    hl1 = jnp.pad(hl[:, 1], pad)
    hr0 = jnp.pad(hr[:, 0], pad)
    hr1 = jnp.pad(hr[:, 1], pad)
    degc = jnp.maximum(degfull, 1.0)
    o0, o1 = _sc_l2(hl0, hl1, srcp, dstp, degc, hr0, hr1)
    return jnp.stack([o0[:N], o1[:N]], axis=1)


# speedup vs baseline: 21.5501x; 1.0754x over previous
import functools

import jax
import jax.numpy as jnp
from jax import lax
from jax.experimental import pallas as pl
from jax.experimental.pallas import tpu as pltpu
from jax.experimental.pallas import tpu_sc as plsc

N = 10000
E = 320000
DIN = 128
DH = 64
DOUT = 2

RN = N
RND = 10240
NSC = 2
NTILE = 16
NW = NSC * NTILE
CHUNK = 80
CPT = 125
EPT = CPT * CHUNK
EPT2 = E // NTILE
SCR = RND // NSC
TPR = SCR // NTILE

BR = 2000
GRID = RN // BR


def _tc1_body(x_ref, wl_ref, wr_ref, br_ref, xl_ref, xr_ref):
    xv = x_ref[...]
    xl_ref[...] = jnp.dot(xv, wl_ref[...], preferred_element_type=jnp.float32)
    xr_ref[...] = jnp.dot(xv, wr_ref[...],
                          preferred_element_type=jnp.float32) + br_ref[...]


def _tc1(xp, w1l, w1r, brow):
    return pl.pallas_call(
        _tc1_body,
        grid=(GRID,),
        in_specs=[
            pl.BlockSpec((BR, DIN), lambda i: (i, 0)),
            pl.BlockSpec((DIN, DH), lambda i: (0, 0)),
            pl.BlockSpec((DIN, DH), lambda i: (0, 0)),
            pl.BlockSpec((1, DH), lambda i: (0, 0)),
        ],
        out_specs=[
            pl.BlockSpec((BR, DH), lambda i: (i, 0)),
            pl.BlockSpec((BR, DH), lambda i: (i, 0)),
        ],
        out_shape=[
            jax.ShapeDtypeStruct((RN, DH), jnp.float32),
            jax.ShapeDtypeStruct((RN, DH), jnp.float32),
        ],
    )(xp, w1l, w1r, brow)


def _tcx_body(ei_ref, src_ref, dst_ref):
    ei = ei_ref[...]
    src_ref[...] = ei[0]
    dst_ref[...] = ei[1]


def _tcx(ei):
    return pl.pallas_call(
        _tcx_body,
        out_shape=[
            jax.ShapeDtypeStruct((E,), jnp.int32),
            jax.ShapeDtypeStruct((E,), jnp.int32),
        ],
    )(ei)


def _tc2_body(p_ref, deg_ref, xr_ref, wl_ref, wr_ref, b2_ref,
              hl_ref, hr_ref):
    pv = p_ref[...]
    ssum = pv[0] + pv[1]
    deg = jnp.maximum(deg_ref[...], 1.0)
    h = jnp.maximum(ssum / deg + xr_ref[...], 0.0)
    hl_ref[...] = jnp.dot(h, wl_ref[...], preferred_element_type=jnp.float32)
    hr_ref[...] = jnp.dot(h, wr_ref[...],
                          preferred_element_type=jnp.float32) + b2_ref[...]


def _tc2(p1, degsum, xr, w2l, w2r, b2p):
    return pl.pallas_call(
        _tc2_body,
        grid=(GRID,),
        in_specs=[
            pl.BlockSpec((2, BR, DH), lambda i: (0, i, 0)),
            pl.BlockSpec((BR, 1), lambda i: (i, 0)),
            pl.BlockSpec((BR, DH), lambda i: (i, 0)),
            pl.BlockSpec((DH, DOUT), lambda i: (0, 0)),
            pl.BlockSpec((DH, DOUT), lambda i: (0, 0)),
            pl.BlockSpec((1, DOUT), lambda i: (0, 0)),
        ],
        out_specs=[
            pl.BlockSpec((BR, DOUT), lambda i: (i, 0)),
            pl.BlockSpec((BR, DOUT), lambda i: (i, 0)),
        ],
        out_shape=[
            jax.ShapeDtypeStruct((RN, DOUT), jnp.float32),
            jax.ShapeDtypeStruct((RN, DOUT), jnp.float32),
        ],
    )(p1, degsum, xr, w2l, w2r, b2p)


_MESH = plsc.VectorSubcoreMesh(core_axis_name="c", subcore_axis_name="s")
_RPT = RN // NTILE
DRPT = RND // NTILE
NBUF = 5
GROUPS = CPT // NBUF


def _idx(buf, i):
    return buf.at[pl.ds(pl.multiple_of(i * CHUNK, 8), CHUNK)]


@functools.partial(
    pl.kernel,
    mesh=_MESH,
    compiler_params=pltpu.CompilerParams(use_tc_tiling_on_sc=False,
                                        needs_layout_passes=False),
    out_type=[
        jax.ShapeDtypeStruct((NSC, RN, DH), jnp.float32),
        jax.ShapeDtypeStruct((NSC, RND), jnp.float32),
    ],
    scratch_types=[
        pltpu.VMEM((EPT,), jnp.int32),
        pltpu.VMEM((EPT,), jnp.int32),
        pltpu.VMEM((NBUF, CHUNK, DH), jnp.float32),
        pltpu.VMEM((RND,), jnp.float32),
        pltpu.VMEM((NTILE, DRPT), jnp.float32),
        pltpu.VMEM((DRPT,), jnp.float32),
        pltpu.VMEM_SHARED((RN, DH), jnp.float32),
        pltpu.VMEM_SHARED((NTILE, RND), jnp.float32),
        pltpu.SemaphoreType.DMA,
        pltpu.SemaphoreType.DMA,
    ],
)
def _sc_agg1(feat, srcs, dsts, zeros64, out, outdeg,
             srcb, dstb, rows, degv, rbuf, dout, acc, dstage, gsem, ssem):
    c = lax.axis_index("c")
    s = lax.axis_index("s")
    wid = s * NSC + c
    pltpu.sync_copy(zeros64.at[pl.ds(s * _RPT, _RPT)],
                    acc.at[pl.ds(s * _RPT, _RPT)])
    pltpu.sync_copy(srcs.at[pl.ds(wid * EPT, EPT)], srcb)
    pltpu.sync_copy(dsts.at[pl.ds(wid * EPT, EPT)], dstb)
    z16 = jnp.zeros((16,), jnp.float32)

    def zbody(k, carry):
        degv[pl.ds(pl.multiple_of(k * 16, 16), 16)] = z16
        return carry

    lax.fori_loop(0, RND // 16, zbody, 0)
    ones_v = jnp.ones((16,), jnp.float32)

    for b in range(NBUF):
        pltpu.async_copy(feat.at[_idx(srcb, b)], rows.at[b], gsem)
    plsc.subcore_barrier()

    def outer(gi, carry):
        for b in range(NBUF):
            i = gi * NBUF + b
            pltpu.make_async_copy(feat.at[_idx(srcb, i)], rows.at[b],
                                  gsem).wait()
            d = pltpu.async_copy(rows.at[b], acc.at[_idx(dstb, i)], ssem,
                                 add=True)
            for k in range(CHUNK // 16):
                off = pl.multiple_of(i * CHUNK + k * 16, 16)
                dv = dstb[pl.ds(off, 16)]
                plsc.addupdate_scatter(degv, [dv], ones_v)
            d.wait()

            @pl.when(i + NBUF < CPT)
            def _():
                pltpu.async_copy(feat.at[_idx(srcb, i + NBUF)], rows.at[b],
                                 gsem)
        return carry

    lax.fori_loop(0, GROUPS, outer, 0)
    pltpu.sync_copy(degv, dstage.at[s])
    plsc.subcore_barrier()
    pltpu.sync_copy(acc.at[pl.ds(s * _RPT, _RPT)],
                    out.at[c, pl.ds(s * _RPT, _RPT)])
    pltpu.sync_copy(dstage.at[:, pl.ds(s * DRPT, DRPT)], rbuf)

    def rbody(k, carry):
        o = pl.multiple_of(k * 16, 16)
        v = rbuf[0, pl.ds(o, 16)]
        for j in range(1, NTILE):
            v = v + rbuf[j, pl.ds(o, 16)]
        dout[pl.ds(o, 16)] = v
        return carry

    lax.fori_loop(0, DRPT // 16, rbody, 0)
    pltpu.sync_copy(dout, outdeg.at[c, pl.ds(s * DRPT, DRPT)])


@functools.partial(
    pl.kernel,
    mesh=_MESH,
    compiler_params=pltpu.CompilerParams(use_tc_tiling_on_sc=False,
                                        needs_layout_passes=False),
    out_type=[
        jax.ShapeDtypeStruct((RND,), jnp.float32),
        jax.ShapeDtypeStruct((RND,), jnp.float32),
    ],
    scratch_types=[
        pltpu.VMEM((EPT2,), jnp.int32),
        pltpu.VMEM((EPT2,), jnp.int32),
        pltpu.VMEM((RND,), jnp.float32),
        pltpu.VMEM((RND,), jnp.float32),
        pltpu.VMEM((SCR,), jnp.float32),
        pltpu.VMEM((SCR,), jnp.float32),
        pltpu.VMEM((NTILE, TPR), jnp.float32),
        pltpu.VMEM((NTILE, TPR), jnp.float32),
        pltpu.VMEM((TPR,), jnp.float32),
        pltpu.VMEM((TPR,), jnp.float32),
        pltpu.VMEM((TPR,), jnp.float32),
        pltpu.VMEM((TPR,), jnp.float32),
        pltpu.VMEM((TPR,), jnp.float32),
        pltpu.VMEM_SHARED((NTILE, 2, SCR), jnp.float32),
    ],
)
def _sc_l2(hl0, hl1, srcs, dsts, deg, hr0, hr1, out0, out1,
           srcb, dstb, hl0b, hl1b, acc0, acc1, rbuf0, rbuf1,
           hr0b, hr1b, degb, dout0, dout1, sstage):
    c = lax.axis_index("c")
    s = lax.axis_index("s")
    lo = c * SCR
    base = c * SCR + s * TPR
    pltpu.sync_copy(srcs.at[pl.ds(s * EPT2, EPT2)], srcb)
    pltpu.sync_copy(dsts.at[pl.ds(s * EPT2, EPT2)], dstb)
    pltpu.sync_copy(hl0, hl0b)
    pltpu.sync_copy(hl1, hl1b)
    pltpu.sync_copy(deg.at[pl.ds(base, TPR)], degb)
    pltpu.sync_copy(hr0.at[pl.ds(base, TPR)], hr0b)
    pltpu.sync_copy(hr1.at[pl.ds(base, TPR)], hr1b)
    z16 = jnp.zeros((16,), jnp.float32)

    def zbody(k, carry):
        o = pl.multiple_of(k * 16, 16)
        acc0[pl.ds(o, 16)] = z16
        acc1[pl.ds(o, 16)] = z16
        return carry

    lax.fori_loop(0, SCR // 16, zbody, 0)

    def body(k, carry):
        off = pl.multiple_of(k * 16, 16)
        sv = srcb[pl.ds(off, 16)]
        dv = dstb[pl.ds(off, 16)]
        m = jnp.logical_and(dv >= lo, dv < lo + SCR)
        dl = jnp.where(m, dv - lo, 0)
        g0 = plsc.load_gather(hl0b, [sv])
        g1 = plsc.load_gather(hl1b, [sv])
        plsc.addupdate_scatter(acc0, [dl], g0, mask=m)
        plsc.addupdate_scatter(acc1, [dl], g1, mask=m)
        return carry

    lax.fori_loop(0, EPT2 // 16, body, 0)
    pltpu.sync_copy(acc0, sstage.at[s, 0])
    pltpu.sync_copy(acc1, sstage.at[s, 1])
    plsc.subcore_barrier()
    pltpu.sync_copy(sstage.at[:, 0, pl.ds(s * TPR, TPR)], rbuf0)
    pltpu.sync_copy(sstage.at[:, 1, pl.ds(s * TPR, TPR)], rbuf1)

    def rbody(k, carry):
        o = pl.multiple_of(k * 16, 16)
        v0 = rbuf0[0, pl.ds(o, 16)]
        v1 = rbuf1[0, pl.ds(o, 16)]
        for j in range(1, NTILE):
            v0 = v0 + rbuf0[j, pl.ds(o, 16)]
            v1 = v1 + rbuf1[j, pl.ds(o, 16)]
        d = degb[pl.ds(o, 16)]
        dout0[pl.ds(o, 16)] = v0 / d + hr0b[pl.ds(o, 16)]
        dout1[pl.ds(o, 16)] = v1 / d + hr1b[pl.ds(o, 16)]
        return carry

    lax.fori_loop(0, TPR // 16, rbody, 0)
    pltpu.sync_copy(dout0, out0.at[pl.ds(base, TPR)])
    pltpu.sync_copy(dout1, out1.at[pl.ds(base, TPR)])


def kernel(x, edge_index, W1_l, W1_r, b1, W2_l, W2_r, b2):
    srcp, dstp = _tcx(edge_index.astype(jnp.int32))
    xl, xr = _tc1(x, W1_l, W1_r, b1[None, :])

    zeros64 = jnp.zeros((RN, DH), jnp.float32)
    p1, pdeg = _sc_agg1(xl, srcp, dstp, zeros64)

    degfull = pdeg[0] + pdeg[1]
    hl, hr = _tc2(p1, degfull[:RN, None], xr, W2_l, W2_r, b2[None, :])

    pad = (0, RND - N)
    hl0 = jnp.pad(hl[:, 0], pad)
    hl1 = jnp.pad(hl[:, 1], pad)
    hr0 = jnp.pad(hr[:, 0], pad)
    hr1 = jnp.pad(hr[:, 1], pad)
    degc = jnp.maximum(degfull, 1.0)
    o0, o1 = _sc_l2(hl0, hl1, srcp, dstp, degc, hr0, hr1)
    return jnp.stack([o0[:N], o1[:N]], axis=1)
